# baseline (speedup 1.0000x reference)
import functools

import jax
import jax.numpy as jnp
from jax import lax
from jax.experimental import pallas as pl
from jax.experimental.pallas import tpu as pltpu
from jax.experimental.pallas import tpu_sc as plsc

NUM_EMBEDDINGS = 1_000_000
EMB_DIM = 32
BATCH_B = 16384
SEQ_T = 50
NUM_CORES = 2
NUM_SUBCORES = 16
NUM_WORKERS = NUM_CORES * NUM_SUBCORES
PER_B = BATCH_B // NUM_WORKERS
GROUP_B = 16
NUM_GROUPS = PER_B // GROUP_B
NBUF = 2

_mesh = plsc.VectorSubcoreMesh(core_axis_name="c", subcore_axis_name="s")


@functools.partial(
    pl.kernel,
    mesh=_mesh,
    out_type=jax.ShapeDtypeStruct((SEQ_T, EMB_DIM, BATCH_B), jnp.float32),
    scratch_types=[
        pltpu.VMEM((NBUF, GROUP_B, SEQ_T), jnp.int32),
        pltpu.VMEM((NBUF, GROUP_B, SEQ_T, EMB_DIM), jnp.float32),
        pltpu.VMEM((NBUF, SEQ_T, EMB_DIM, GROUP_B), jnp.float32),
        pltpu.SemaphoreType.DMA((NBUF,)),
        pltpu.SemaphoreType.DMA((NBUF,)),
    ],
    compiler_params=pltpu.CompilerParams(
        use_tc_tiling_on_sc=False, needs_layout_passes=False
    ),
)
def _gather_kernel(ids_hbm, table_hbm, out_hbm, idx_v, gbuf, tbuf, gsem, wsem):
    wid = lax.axis_index("s") * NUM_CORES + lax.axis_index("c")
    b_base = wid * PER_B
    lane = lax.iota(jnp.int32, 16)

    def body(g, carry):
        p = lax.rem(g, NBUF)
        b0 = b_base + g * GROUP_B

        pltpu.sync_copy(ids_hbm.at[pl.ds(b0, GROUP_B)], idx_v.at[p])
        for k in range(GROUP_B):
            pltpu.async_copy(
                table_hbm.at[idx_v.at[p].at[k]], gbuf.at[p].at[k], gsem.at[p]
            )
        for k in range(GROUP_B):
            pltpu.make_async_copy(
                table_hbm.at[idx_v.at[p].at[k]], gbuf.at[p].at[k], gsem.at[p]
            ).wait()

        @pl.when(g >= NBUF)
        def _():
            pltpu.make_async_copy(
                tbuf.at[p], out_hbm.at[:, :, pl.ds(0, GROUP_B)], wsem.at[p]
            ).wait()

        def tbody(t, tc):
            tvec = jnp.full((16,), 0, jnp.int32) + t
            for d in range(EMB_DIM):
                dvec = jnp.full((16,), d, jnp.int32)
                v = plsc.load_gather(gbuf.at[p], [lane, tvec, dvec])
                tbuf[p, t, d, :] = v
            return tc

        lax.fori_loop(0, SEQ_T, tbody, 0)

        pltpu.async_copy(
            tbuf.at[p], out_hbm.at[:, :, pl.ds(b0, GROUP_B)], wsem.at[p]
        )
        return carry

    lax.fori_loop(0, NUM_GROUPS, body, 0)

    for p in range(NBUF):
        pltpu.make_async_copy(
            tbuf.at[p], out_hbm.at[:, :, pl.ds(0, GROUP_B)], wsem.at[p]
        ).wait()


@jax.jit
def kernel(ids, emb_matrix):
    out_t = _gather_kernel(ids.astype(jnp.int32), emb_matrix)
    return jnp.transpose(out_t, (2, 0, 1))


# speedup vs baseline: 1.2303x; 1.2303x over previous
import functools

import jax
import jax.numpy as jnp
from jax import lax
from jax.experimental import pallas as pl
from jax.experimental.pallas import tpu as pltpu
from jax.experimental.pallas import tpu_sc as plsc

NUM_EMBEDDINGS = 1_000_000
EMB_DIM = 32
BATCH_B = 16384
SEQ_T = 50
NUM_CORES = 2
NUM_SUBCORES = 16
NUM_WORKERS = NUM_CORES * NUM_SUBCORES
PER_B = BATCH_B // NUM_WORKERS
GROUP_B = 16
NUM_GROUPS = PER_B // GROUP_B
NBUF = 2

_mesh = plsc.VectorSubcoreMesh(core_axis_name="c", subcore_axis_name="s")


@functools.partial(
    pl.kernel,
    mesh=_mesh,
    out_type=jax.ShapeDtypeStruct((SEQ_T, EMB_DIM, BATCH_B), jnp.float32),
    scratch_types=[
        pltpu.VMEM((NBUF, GROUP_B, SEQ_T), jnp.int32),
        pltpu.VMEM((NBUF, GROUP_B, SEQ_T, EMB_DIM), jnp.float32),
        pltpu.VMEM((NBUF, SEQ_T, EMB_DIM, GROUP_B + 1), jnp.float32),
        pltpu.SemaphoreType.DMA((NBUF,)),
        pltpu.SemaphoreType.DMA((NBUF,)),
    ],
    compiler_params=pltpu.CompilerParams(
        use_tc_tiling_on_sc=False, needs_layout_passes=False
    ),
)
def _gather_kernel(ids_hbm, table_hbm, out_hbm, idx_v, gbuf, tbuf, gsem, wsem):
    wid = lax.axis_index("s") * NUM_CORES + lax.axis_index("c")
    b_base = wid * PER_B
    lane = lax.iota(jnp.int32, 16)

    def body(g, carry):
        p = lax.rem(g, NBUF)
        b0 = b_base + g * GROUP_B

        pltpu.sync_copy(ids_hbm.at[pl.ds(b0, GROUP_B)], idx_v.at[p])
        for k in range(GROUP_B):
            pltpu.async_copy(
                table_hbm.at[idx_v.at[p].at[k]], gbuf.at[p].at[k], gsem.at[p]
            )
        for k in range(GROUP_B):
            pltpu.make_async_copy(
                table_hbm.at[idx_v.at[p].at[k]], gbuf.at[p].at[k], gsem.at[p]
            ).wait()

        @pl.when(g >= NBUF)
        def _():
            pltpu.make_async_copy(
                tbuf.at[p].at[:, :, pl.ds(0, GROUP_B)],
                out_hbm.at[:, :, pl.ds(0, GROUP_B)],
                wsem.at[p],
            ).wait()

        def tbody(t, tc):
            tvec = jnp.full((16,), 0, jnp.int32) + t
            for k in range(GROUP_B):
                kvec = jnp.full((16,), k, jnp.int32)
                for dh in range(EMB_DIM // 16):
                    v = gbuf[p, k, t, pl.ds(dh * 16, 16)]
                    plsc.store_scatter(
                        tbuf.at[p], [tvec, lane + dh * 16, kvec], v
                    )
            return tc

        lax.fori_loop(0, SEQ_T, tbody, 0)

        pltpu.async_copy(
            tbuf.at[p].at[:, :, pl.ds(0, GROUP_B)],
            out_hbm.at[:, :, pl.ds(b0, GROUP_B)],
            wsem.at[p],
        )
        return carry

    lax.fori_loop(0, NUM_GROUPS, body, 0)

    for p in range(NBUF):
        pltpu.make_async_copy(
            tbuf.at[p].at[:, :, pl.ds(0, GROUP_B)],
            out_hbm.at[:, :, pl.ds(0, GROUP_B)],
            wsem.at[p],
        ).wait()


@jax.jit
def kernel(ids, emb_matrix):
    out_t = _gather_kernel(ids.astype(jnp.int32), emb_matrix)
    return jnp.transpose(out_t, (2, 0, 1))


# speedup vs baseline: 1.4358x; 1.1670x over previous
import functools

import jax
import jax.numpy as jnp
from jax import lax
from jax.experimental import pallas as pl
from jax.experimental.pallas import tpu as pltpu
from jax.experimental.pallas import tpu_sc as plsc

NUM_EMBEDDINGS = 1_000_000
EMB_DIM = 32
BATCH_B = 16384
SEQ_T = 50
NUM_CORES = 2
NUM_SUBCORES = 16
NUM_WORKERS = NUM_CORES * NUM_SUBCORES
PER_B = BATCH_B // NUM_WORKERS
GROUP_B = 16
NUM_GROUPS = PER_B // GROUP_B
NBUF = 2

_mesh = plsc.VectorSubcoreMesh(core_axis_name="c", subcore_axis_name="s")


@functools.partial(
    pl.kernel,
    mesh=_mesh,
    out_type=jax.ShapeDtypeStruct((SEQ_T, EMB_DIM, BATCH_B), jnp.float32),
    scratch_types=[
        pltpu.VMEM((NBUF, GROUP_B, SEQ_T), jnp.int32),
        pltpu.VMEM((NBUF, GROUP_B, SEQ_T, EMB_DIM), jnp.float32),
        pltpu.VMEM((NBUF, SEQ_T, EMB_DIM, GROUP_B + 1), jnp.float32),
        pltpu.SemaphoreType.DMA((NBUF,)),
        pltpu.SemaphoreType.DMA((NBUF,)),
    ],
    compiler_params=pltpu.CompilerParams(
        use_tc_tiling_on_sc=False, needs_layout_passes=False
    ),
)
def _gather_kernel(ids_hbm, table_hbm, out_hbm, idx_v, gbuf, tbuf, gsem, wsem):
    wid = lax.axis_index("s") * NUM_CORES + lax.axis_index("c")
    b_base = wid * PER_B
    lane = lax.iota(jnp.int32, 16)

    def body(g, carry):
        p = lax.rem(g, NBUF)
        b0 = b_base + g * GROUP_B

        pltpu.sync_copy(ids_hbm.at[pl.ds(b0, GROUP_B)], idx_v.at[p])
        for k in range(GROUP_B):
            pltpu.async_copy(
                table_hbm.at[idx_v.at[p].at[k]], gbuf.at[p].at[k], gsem.at[p]
            )
        for k in range(GROUP_B):
            pltpu.make_async_copy(
                table_hbm.at[idx_v.at[p].at[k]], gbuf.at[p].at[k], gsem.at[p]
            ).wait()

        @pl.when(g >= NBUF)
        def _():
            pltpu.make_async_copy(
                tbuf.at[p].at[:, :, pl.ds(0, GROUP_B)],
                out_hbm.at[:, :, pl.ds(0, GROUP_B)],
                wsem.at[p],
            ).wait()

        @plsc.parallel_loop(0, SEQ_T, unroll=2)
        def tbody(t):
            tvec = jnp.full((16,), 0, jnp.int32) + t
            for k in range(GROUP_B):
                kvec = jnp.full((16,), k, jnp.int32)
                for dh in range(EMB_DIM // 16):
                    v = gbuf[p, k, t, pl.ds(dh * 16, 16)]
                    plsc.store_scatter(
                        tbuf.at[p], [tvec, lane + dh * 16, kvec], v
                    )

        pltpu.async_copy(
            tbuf.at[p].at[:, :, pl.ds(0, GROUP_B)],
            out_hbm.at[:, :, pl.ds(b0, GROUP_B)],
            wsem.at[p],
        )
        return carry

    lax.fori_loop(0, NUM_GROUPS, body, 0)

    for p in range(NBUF):
        pltpu.make_async_copy(
            tbuf.at[p].at[:, :, pl.ds(0, GROUP_B)],
            out_hbm.at[:, :, pl.ds(0, GROUP_B)],
            wsem.at[p],
        ).wait()


@jax.jit
def kernel(ids, emb_matrix):
    out_t = _gather_kernel(ids.astype(jnp.int32), emb_matrix)
    return jnp.transpose(out_t, (2, 0, 1))


# speedup vs baseline: 1.4605x; 1.0172x over previous
import functools

import jax
import jax.numpy as jnp
from jax import lax
from jax.experimental import pallas as pl
from jax.experimental.pallas import tpu as pltpu
from jax.experimental.pallas import tpu_sc as plsc

NUM_EMBEDDINGS = 1_000_000
EMB_DIM = 32
BATCH_B = 16384
SEQ_T = 50
NUM_CORES = 2
NUM_SUBCORES = 16
NUM_WORKERS = NUM_CORES * NUM_SUBCORES
PER_B = BATCH_B // NUM_WORKERS
GROUP_B = 16
NUM_GROUPS = PER_B // GROUP_B
NBUF = 2

_mesh = plsc.VectorSubcoreMesh(core_axis_name="c", subcore_axis_name="s")


@functools.partial(
    pl.kernel,
    mesh=_mesh,
    out_type=jax.ShapeDtypeStruct((SEQ_T, EMB_DIM, BATCH_B), jnp.float32),
    scratch_types=[
        pltpu.VMEM((NBUF, GROUP_B, SEQ_T), jnp.int32),
        pltpu.VMEM((NBUF, GROUP_B, SEQ_T, EMB_DIM), jnp.float32),
        pltpu.VMEM((NBUF, SEQ_T, EMB_DIM, GROUP_B + 1), jnp.float32),
        pltpu.SemaphoreType.DMA((NBUF,)),
        pltpu.SemaphoreType.DMA((NBUF,)),
    ],
    compiler_params=pltpu.CompilerParams(
        use_tc_tiling_on_sc=False, needs_layout_passes=False
    ),
)
def _gather_kernel(ids_hbm, table_hbm, out_hbm, idx_v, gbuf, tbuf, gsem, wsem):
    wid = lax.axis_index("s") * NUM_CORES + lax.axis_index("c")
    b_base = wid * PER_B
    lane = lax.iota(jnp.int32, 16)

    def fire_group(g):
        p = lax.rem(g, NBUF)
        b0 = b_base + g * GROUP_B
        pltpu.sync_copy(ids_hbm.at[pl.ds(b0, GROUP_B)], idx_v.at[p])
        for k in range(GROUP_B):
            pltpu.async_copy(
                table_hbm.at[idx_v.at[p].at[k]], gbuf.at[p].at[k], gsem.at[p]
            )

    fire_group(0)

    def body(g, carry):
        p = lax.rem(g, NBUF)
        b0 = b_base + g * GROUP_B

        @pl.when(g + 1 < NUM_GROUPS)
        def _():
            fire_group(g + 1)

        for k in range(GROUP_B):
            pltpu.make_async_copy(
                table_hbm.at[idx_v.at[p].at[k]], gbuf.at[p].at[k], gsem.at[p]
            ).wait()

        @pl.when(g >= NBUF)
        def _():
            pltpu.make_async_copy(
                tbuf.at[p].at[:, :, pl.ds(0, GROUP_B)],
                out_hbm.at[:, :, pl.ds(0, GROUP_B)],
                wsem.at[p],
            ).wait()

        @plsc.parallel_loop(0, SEQ_T, unroll=4)
        def tbody(t):
            tvec = jnp.full((16,), 0, jnp.int32) + t
            for k in range(GROUP_B):
                kvec = jnp.full((16,), k, jnp.int32)
                for dh in range(EMB_DIM // 16):
                    v = gbuf[p, k, t, pl.ds(dh * 16, 16)]
                    plsc.store_scatter(
                        tbuf.at[p], [tvec, lane + dh * 16, kvec], v
                    )

        pltpu.async_copy(
            tbuf.at[p].at[:, :, pl.ds(0, GROUP_B)],
            out_hbm.at[:, :, pl.ds(b0, GROUP_B)],
            wsem.at[p],
        )
        return carry

    lax.fori_loop(0, NUM_GROUPS, body, 0)

    for p in range(NBUF):
        pltpu.make_async_copy(
            tbuf.at[p].at[:, :, pl.ds(0, GROUP_B)],
            out_hbm.at[:, :, pl.ds(0, GROUP_B)],
            wsem.at[p],
        ).wait()


@jax.jit
def kernel(ids, emb_matrix):
    out_t = _gather_kernel(ids.astype(jnp.int32), emb_matrix)
    return jnp.transpose(out_t, (2, 0, 1))
